# baseline (speedup 1.0000x reference)
import functools

import jax
import jax.numpy as jnp
from jax import lax
from jax.experimental import pallas as pl
from jax.experimental.pallas import tpu as pltpu
from jax.experimental.pallas import tpu_sc as plsc

C = 1074
K = 5
D = 2048
N = 16384

NC, NS = 2, 16
NW = NC * NS
CPAD = 1280
CPT = CPAD // NW
SPC = CPAD // NS
DH = D // NC
RPS = N // NS
RA = 128
TCOLS = D // NW
RPW = N // NW
RC = 16
CB = 8

def _segment_sums_body(feat_hbm, lab_hbm, out_hbm, feat_v, lab_v, acc_v):
    cidx = lax.axis_index("c")
    s = lax.axis_index("s")
    wid = s * NC + cidx
    half = (wid % 2) * TCOLS

    def z(i, _):
        acc_v[pl.ds(i * 16, 16)] = jnp.zeros((16,), jnp.float32)
        return 0

    lax.fori_loop(0, CPAD * TCOLS // 16, z, 0)

    def chunk(jc, _):
        rbase = jc * RA
        pltpu.sync_copy(
            feat_hbm.at[pl.ds(rbase, RA), pl.ds((wid // 2) * 128, 128)], feat_v)
        pltpu.sync_copy(lab_hbm.at[pl.ds(rbase, RA)], lab_v)

        def grp(q, _):
            lv = lab_v[pl.ds(q * 16, 16)]
            for rr in range(16):
                base = lv[rr] * TCOLS
                for u in range(TCOLS // 16):
                    x = feat_v[q * 16 + rr, pl.ds(half + u * 16, 16)]
                    plsc.addupdate(acc_v.at[pl.ds(base + u * 16, 16)], x)
            return 0

        lax.fori_loop(0, RA // 16, grp, 0)
        return 0

    lax.fori_loop(0, N // RA, chunk, 0)
    pltpu.sync_copy(acc_v, out_hbm.at[wid])


def _means_body(lab_ref, sums_ref, means_ref):
    i = pl.program_id(0)
    lab = lab_ref[...]
    for t in range(CB):
        cid = i * CB + t
        cnt = jnp.sum(jnp.where(lab == cid, 1, 0))
        denom = jnp.maximum(cnt, 1).astype(jnp.float32)
        means_ref[pl.ds(t, 1), :] = sums_ref[pl.ds(t, 1), :] / denom


def _means_call(lab8, sums):
    return pl.pallas_call(
        _means_body,
        grid=(CPAD // CB,),
        in_specs=[
            pl.BlockSpec((8, N // 8), lambda i: (0, 0)),
            pl.BlockSpec((CB, D), lambda i: (i, 0)),
        ],
        out_specs=pl.BlockSpec((CB, D), lambda i: (i, 0)),
        out_shape=jax.ShapeDtypeStruct((CPAD, D), jnp.float32),
    )(lab8, sums)


def _dists_body(feat_hbm, lab_hbm, means_hbm, out_hbm, feat_v, mean_v, lab_v, ssq_v, sem):
    cidx = lax.axis_index("c")
    s = lax.axis_index("s")
    wid = s * NC + cidx

    lanes = lax.iota(jnp.int32, 16)

    def chunk(j, _):
        base = wid * RPW + j * RC
        pltpu.sync_copy(lab_hbm.at[pl.ds(base, RC)], lab_v)
        pltpu.sync_copy(feat_hbm.at[pl.ds(base, RC)], feat_v)
        pltpu.async_copy(means_hbm.at[lab_v], mean_v, sem).wait()

        def colblk(b, acc_out):
            def colstep(k2, acc_in):
                for u in range(8):
                    col = b * 128 + k2 * 8 + u
                    ci = jnp.full((16,), col, jnp.int32)
                    f = plsc.load_gather(feat_v, [lanes, ci])
                    m = plsc.load_gather(mean_v, [lanes, ci])
                    dd = f - m
                    acc_in = acc_in + dd * dd
                return acc_in

            acc_in = lax.fori_loop(0, 16, colstep,
                                   jnp.zeros((16,), jnp.float32))
            return acc_out + acc_in

        acc = lax.fori_loop(0, 16, colblk, jnp.zeros((16,), jnp.float32))
        ssq_v[pl.ds(j * RC, RC)] = acc
        return 0

    lax.fori_loop(0, RPW // RC, chunk, 0)
    pltpu.sync_copy(ssq_v, out_hbm.at[pl.ds(wid * RPW, RPW)])


def _topk_body(lab_ref, ssq_ref, idx_ref, w_ref):
    i = pl.program_id(0)
    lab = lab_ref[...]
    d = jnp.sqrt(ssq_ref[...] + 1e-12)
    r8 = lax.broadcasted_iota(jnp.int32, (8, N // 8), 0)
    c8 = lax.broadcasted_iota(jnp.int32, (8, N // 8), 1)
    flat = r8 * (N // 8) + c8
    col128 = lax.broadcasted_iota(jnp.int32, (1, 128), 1)
    for t in range(CB):
        cid = i * CB + t
        member = lab == cid
        cnt = jnp.sum(member.astype(jnp.int32))
        cntf = jnp.maximum(cnt, 1).astype(jnp.float32)
        dist = jnp.where(member, d, jnp.inf)
        idx_row = jnp.zeros((1, 128), jnp.int32)
        w_row = jnp.zeros((1, 128), jnp.float32)
        for k in range(K):
            m = jnp.min(dist)
            am = jnp.min(jnp.where(dist == m, flat, N))
            wk = jnp.where(cnt >= K, jnp.float32(1.0 / K),
                           jnp.where(k < cnt, 1.0 / cntf, jnp.float32(0.0)))
            if k == 0:
                idx_row = jnp.broadcast_to(am, (1, 128)).astype(jnp.int32)
            else:
                idx_row = jnp.where(col128 == k, am, idx_row)
            w_row = jnp.where(col128 == k, wk, w_row)
            dist = jnp.where(flat == am, jnp.inf, dist)
        idx_ref[pl.ds(t, 1), :] = idx_row
        w_ref[pl.ds(t, 1), :] = w_row


def _topk_call(lab8, ssq8):
    return pl.pallas_call(
        _topk_body,
        grid=(CPAD // CB,),
        in_specs=[
            pl.BlockSpec((8, N // 8), lambda i: (0, 0)),
            pl.BlockSpec((8, N // 8), lambda i: (0, 0)),
        ],
        out_specs=[
            pl.BlockSpec((CB, 128), lambda i: (i, 0)),
            pl.BlockSpec((CB, 128), lambda i: (i, 0)),
        ],
        out_shape=[
            jax.ShapeDtypeStruct((CPAD, 128), jnp.int32),
            jax.ShapeDtypeStruct((CPAD, 128), jnp.float32),
        ],
    )(lab8, ssq8)


def _protos_body(feat_hbm, idx_hbm, w_hbm, out_hbm,
                 idx_v, w_v, rows_v, proto8_v, sem):
    cidx = lax.axis_index("c")
    s = lax.axis_index("s")
    wid = s * NC + cidx
    base = wid * CPT
    pltpu.sync_copy(idx_hbm.at[pl.ds(base, CPT)], idx_v)
    pltpu.sync_copy(w_hbm.at[pl.ds(base, CPT)], w_v)

    def grp_step(g, _):
        def cls_step(t, _):
            tl = g * 8 + t
            pltpu.async_copy(
                feat_hbm.at[idx_v.at[tl, pl.ds(0, 8)]], rows_v, sem).wait()
            wvec = w_v[tl, pl.ds(0, 16)]
            w0 = wvec[0]
            w1 = wvec[1]
            w2 = wvec[2]
            w3 = wvec[3]
            w4 = wvec[4]

            def col(kk, _):
                sl = pl.ds(kk * 16, 16)
                acc = rows_v[0, sl] * w0
                acc = acc + rows_v[1, sl] * w1
                acc = acc + rows_v[2, sl] * w2
                acc = acc + rows_v[3, sl] * w3
                acc = acc + rows_v[4, sl] * w4
                proto8_v[t, sl] = acc
                return 0

            lax.fori_loop(0, D // 16, col, 0)
            return 0

        lax.fori_loop(0, 8, cls_step, 0)
        pltpu.sync_copy(proto8_v, out_hbm.at[pl.ds(base + g * 8, 8)])
        return 0

    lax.fori_loop(0, CPT // 8, grp_step, 0)


@functools.lru_cache(maxsize=1)
def _build_sc_kernels():
    mesh = plsc.VectorSubcoreMesh(
        core_axis_name="c", subcore_axis_name="s",
        num_cores=NC, num_subcores=NS)
    params = pltpu.CompilerParams(needs_layout_passes=False)
    segment_sums = pl.kernel(
        _segment_sums_body,
        out_type=jax.ShapeDtypeStruct((NW, CPAD * TCOLS), jnp.float32),
        mesh=mesh,
        compiler_params=params,
        scratch_types=[
            pltpu.VMEM((RA, 128), jnp.float32),
            pltpu.VMEM((RA,), jnp.int32),
            pltpu.VMEM((CPAD * TCOLS,), jnp.float32),
        ],
    )
    dists = pl.kernel(
        _dists_body,
        out_type=jax.ShapeDtypeStruct((N,), jnp.float32),
        mesh=mesh,
        compiler_params=params,
        scratch_types=[
            pltpu.VMEM((RC, D), jnp.float32),
            pltpu.VMEM((RC, D), jnp.float32),
            pltpu.VMEM((RC,), jnp.int32),
            pltpu.VMEM((RPW,), jnp.float32),
            pltpu.SemaphoreType.DMA,
        ],
    )
    protos_k = pl.kernel(
        _protos_body,
        out_type=jax.ShapeDtypeStruct((CPAD, D), jnp.float32),
        mesh=mesh,
        compiler_params=params,
        scratch_types=[
            pltpu.VMEM((CPT, 128), jnp.int32),
            pltpu.VMEM((CPT, 128), jnp.float32),
            pltpu.VMEM((8, D), jnp.float32),
            pltpu.VMEM((8, D), jnp.float32),
            pltpu.SemaphoreType.DMA,
        ],
    )
    return segment_sums, dists, protos_k


def kernel(features, labels):
    segment_sums, dists, protos_k = _build_sc_kernels()
    labels = labels.astype(jnp.int32)
    lab8 = labels.reshape(8, N // 8)
    sums3 = segment_sums(features, labels)
    sums = sums3.reshape(NW, CPAD, TCOLS).transpose(1, 0, 2).reshape(CPAD, D)
    means = _means_call(lab8, sums)
    ssq = dists(features, labels, means)
    idx128, w128 = _topk_call(lab8, ssq.reshape(8, N // 8))
    protos = protos_k(features, idx128, w128)
    return protos[:C]


# speedup vs baseline: 1.3293x; 1.3293x over previous
import functools

import jax
import jax.numpy as jnp
from jax import lax
from jax.experimental import pallas as pl
from jax.experimental.pallas import tpu as pltpu
from jax.experimental.pallas import tpu_sc as plsc

C = 1074
K = 5
D = 2048
N = 16384

NC, NS = 2, 16
NW = NC * NS
CPAD = 1280
CPT = CPAD // NW
SPC = CPAD // NS
DH = D // NC
RPS = N // NS
RA = 128
TCOLS = D // NW
RPW = N // NW
RC = 16
CB = 8

def _segment_sums_body(feat_hbm, lab_hbm, out_hbm, feat_v, lab_v, acc_v):
    cidx = lax.axis_index("c")
    s = lax.axis_index("s")
    wid = s * NC + cidx
    half = (wid % 2) * TCOLS

    def z(i, _):
        acc_v[pl.ds(i * 16, 16)] = jnp.zeros((16,), jnp.float32)
        return 0

    lax.fori_loop(0, CPAD * TCOLS // 16, z, 0)

    def chunk(jc, _):
        rbase = jc * RA
        pltpu.sync_copy(
            feat_hbm.at[pl.ds(rbase, RA), pl.ds((wid // 2) * 128, 128)], feat_v)
        pltpu.sync_copy(lab_hbm.at[pl.ds(rbase, RA)], lab_v)

        def grp(q, _):
            lv = lab_v[pl.ds(q * 16, 16)]
            for rr in range(16):
                base = lv[rr] * TCOLS
                for u in range(TCOLS // 16):
                    x = feat_v[q * 16 + rr, pl.ds(half + u * 16, 16)]
                    plsc.addupdate(acc_v.at[pl.ds(base + u * 16, 16)], x)
            return 0

        lax.fori_loop(0, RA // 16, grp, 0)
        return 0

    lax.fori_loop(0, N // RA, chunk, 0)
    pltpu.sync_copy(acc_v, out_hbm.at[wid])


def _means_body(lab_ref, sums_ref, means_ref):
    i = pl.program_id(0)
    lab = lab_ref[...]
    for t in range(CB):
        cid = i * CB + t
        cnt = jnp.sum(jnp.where(lab == cid, 1, 0))
        denom = jnp.maximum(cnt, 1).astype(jnp.float32)
        means_ref[pl.ds(t, 1), :] = sums_ref[pl.ds(t, 1), :] / denom


def _means_call(lab8, sums):
    return pl.pallas_call(
        _means_body,
        grid=(CPAD // CB,),
        in_specs=[
            pl.BlockSpec((8, N // 8), lambda i: (0, 0)),
            pl.BlockSpec((CB, D), lambda i: (i, 0)),
        ],
        out_specs=pl.BlockSpec((CB, D), lambda i: (i, 0)),
        out_shape=jax.ShapeDtypeStruct((CPAD, D), jnp.float32),
    )(lab8, sums)


def _dists_body(feat_hbm, lab_hbm, means_hbm, out_hbm, feat_v, mean_v, lab_v, ssq_v, sem):
    cidx = lax.axis_index("c")
    s = lax.axis_index("s")
    wid = s * NC + cidx

    lanes = lax.iota(jnp.int32, 16)

    def chunk(j, _):
        base = wid * RPW + j * RC
        pltpu.sync_copy(lab_hbm.at[pl.ds(base, RC)], lab_v)
        pltpu.sync_copy(feat_hbm.at[pl.ds(base, RC)], feat_v)
        pltpu.async_copy(means_hbm.at[lab_v], mean_v, sem).wait()

        def row(r, vec):
            def colstep(k2, acc):
                for u in range(4):
                    sl = pl.ds(k2 * 64 + u * 16, 16)
                    dd = feat_v[r, sl] - mean_v[r, sl]
                    acc = acc + dd * dd
                return acc

            acc = lax.fori_loop(0, D // 64, colstep,
                                jnp.zeros((16,), jnp.float32))
            return jnp.where(lanes == r, jnp.sum(acc), vec)

        vec = lax.fori_loop(0, RC, row, jnp.zeros((16,), jnp.float32))
        ssq_v[pl.ds(j * RC, RC)] = vec
        return 0

    lax.fori_loop(0, RPW // RC, chunk, 0)
    pltpu.sync_copy(ssq_v, out_hbm.at[pl.ds(wid * RPW, RPW)])


def _topk_body(lab_ref, ssq_ref, idx_ref, w_ref):
    i = pl.program_id(0)
    lab = lab_ref[...]
    d = jnp.sqrt(ssq_ref[...] + 1e-12)
    r8 = lax.broadcasted_iota(jnp.int32, (8, N // 8), 0)
    c8 = lax.broadcasted_iota(jnp.int32, (8, N // 8), 1)
    flat = r8 * (N // 8) + c8
    col128 = lax.broadcasted_iota(jnp.int32, (1, 128), 1)
    for t in range(CB):
        cid = i * CB + t
        member = lab == cid
        cnt = jnp.sum(member.astype(jnp.int32))
        cntf = jnp.maximum(cnt, 1).astype(jnp.float32)
        dist = jnp.where(member, d, jnp.inf)
        idx_row = jnp.zeros((1, 128), jnp.int32)
        w_row = jnp.zeros((1, 128), jnp.float32)
        for k in range(K):
            m = jnp.min(dist)
            am = jnp.min(jnp.where(dist == m, flat, N))
            wk = jnp.where(cnt >= K, jnp.float32(1.0 / K),
                           jnp.where(k < cnt, 1.0 / cntf, jnp.float32(0.0)))
            if k == 0:
                idx_row = jnp.broadcast_to(am, (1, 128)).astype(jnp.int32)
            else:
                idx_row = jnp.where(col128 == k, am, idx_row)
            w_row = jnp.where(col128 == k, wk, w_row)
            dist = jnp.where(flat == am, jnp.inf, dist)
        idx_ref[pl.ds(t, 1), :] = idx_row
        w_ref[pl.ds(t, 1), :] = w_row


def _topk_call(lab8, ssq8):
    return pl.pallas_call(
        _topk_body,
        grid=(CPAD // CB,),
        in_specs=[
            pl.BlockSpec((8, N // 8), lambda i: (0, 0)),
            pl.BlockSpec((8, N // 8), lambda i: (0, 0)),
        ],
        out_specs=[
            pl.BlockSpec((CB, 128), lambda i: (i, 0)),
            pl.BlockSpec((CB, 128), lambda i: (i, 0)),
        ],
        out_shape=[
            jax.ShapeDtypeStruct((CPAD, 128), jnp.int32),
            jax.ShapeDtypeStruct((CPAD, 128), jnp.float32),
        ],
    )(lab8, ssq8)


def _protos_body(feat_hbm, idx_hbm, w_hbm, out_hbm,
                 idx_v, w_v, rows_v, proto8_v, sem):
    cidx = lax.axis_index("c")
    s = lax.axis_index("s")
    wid = s * NC + cidx
    base = wid * CPT
    pltpu.sync_copy(idx_hbm.at[pl.ds(base, CPT)], idx_v)
    pltpu.sync_copy(w_hbm.at[pl.ds(base, CPT)], w_v)

    def grp_step(g, _):
        def cls_step(t, _):
            tl = g * 8 + t
            pltpu.async_copy(
                feat_hbm.at[idx_v.at[tl, pl.ds(0, 8)]], rows_v, sem).wait()
            wvec = w_v[tl, pl.ds(0, 16)]
            w0 = wvec[0]
            w1 = wvec[1]
            w2 = wvec[2]
            w3 = wvec[3]
            w4 = wvec[4]

            def col(kk, _):
                sl = pl.ds(kk * 16, 16)
                acc = rows_v[0, sl] * w0
                acc = acc + rows_v[1, sl] * w1
                acc = acc + rows_v[2, sl] * w2
                acc = acc + rows_v[3, sl] * w3
                acc = acc + rows_v[4, sl] * w4
                proto8_v[t, sl] = acc
                return 0

            lax.fori_loop(0, D // 16, col, 0)
            return 0

        lax.fori_loop(0, 8, cls_step, 0)
        pltpu.sync_copy(proto8_v, out_hbm.at[pl.ds(base + g * 8, 8)])
        return 0

    lax.fori_loop(0, CPT // 8, grp_step, 0)


@functools.lru_cache(maxsize=1)
def _build_sc_kernels():
    mesh = plsc.VectorSubcoreMesh(
        core_axis_name="c", subcore_axis_name="s",
        num_cores=NC, num_subcores=NS)
    params = pltpu.CompilerParams(needs_layout_passes=False)
    segment_sums = pl.kernel(
        _segment_sums_body,
        out_type=jax.ShapeDtypeStruct((NW, CPAD * TCOLS), jnp.float32),
        mesh=mesh,
        compiler_params=params,
        scratch_types=[
            pltpu.VMEM((RA, 128), jnp.float32),
            pltpu.VMEM((RA,), jnp.int32),
            pltpu.VMEM((CPAD * TCOLS,), jnp.float32),
        ],
    )
    dists = pl.kernel(
        _dists_body,
        out_type=jax.ShapeDtypeStruct((N,), jnp.float32),
        mesh=mesh,
        compiler_params=params,
        scratch_types=[
            pltpu.VMEM((RC, D), jnp.float32),
            pltpu.VMEM((RC, D), jnp.float32),
            pltpu.VMEM((RC,), jnp.int32),
            pltpu.VMEM((RPW,), jnp.float32),
            pltpu.SemaphoreType.DMA,
        ],
    )
    protos_k = pl.kernel(
        _protos_body,
        out_type=jax.ShapeDtypeStruct((CPAD, D), jnp.float32),
        mesh=mesh,
        compiler_params=params,
        scratch_types=[
            pltpu.VMEM((CPT, 128), jnp.int32),
            pltpu.VMEM((CPT, 128), jnp.float32),
            pltpu.VMEM((8, D), jnp.float32),
            pltpu.VMEM((8, D), jnp.float32),
            pltpu.SemaphoreType.DMA,
        ],
    )
    return segment_sums, dists, protos_k


def kernel(features, labels):
    segment_sums, dists, protos_k = _build_sc_kernels()
    labels = labels.astype(jnp.int32)
    lab8 = labels.reshape(8, N // 8)
    sums3 = segment_sums(features, labels)
    sums = sums3.reshape(NW, CPAD, TCOLS).transpose(1, 0, 2).reshape(CPAD, D)
    means = _means_call(lab8, sums)
    ssq = dists(features, labels, means)
    idx128, w128 = _topk_call(lab8, ssq.reshape(8, N // 8))
    protos = protos_k(features, idx128, w128)
    return protos[:C]


# speedup vs baseline: 3.4889x; 2.6246x over previous
import functools

import jax
import jax.numpy as jnp
from jax import lax
from jax.experimental import pallas as pl
from jax.experimental.pallas import tpu as pltpu
from jax.experimental.pallas import tpu_sc as plsc

C = 1074
K = 5
D = 2048
N = 16384

NC, NS = 2, 16
NW = NC * NS
CPAD = 1280
CPT = CPAD // NW
SPC = CPAD // NS
DH = D // NC
RPS = N // NS
RA = 128
TCOLS = D // NW
RPW = N // NW
RC = 16
CB = 8
CBL = 128

def _segment_sums_body(feat_hbm, lab_hbm, out_hbm, feat_v, lab_v, acc_v):
    cidx = lax.axis_index("c")
    s = lax.axis_index("s")
    wid = s * NC + cidx
    half = (wid % 2) * TCOLS

    def z(i, _):
        acc_v[pl.ds(i * 16, 16)] = jnp.zeros((16,), jnp.float32)
        return 0

    lax.fori_loop(0, CPAD * TCOLS // 16, z, 0)

    def chunk(jc, _):
        rbase = jc * RA
        pltpu.sync_copy(
            feat_hbm.at[pl.ds(rbase, RA), pl.ds((wid // 2) * 128, 128)], feat_v)
        pltpu.sync_copy(lab_hbm.at[pl.ds(rbase, RA)], lab_v)

        def grp(q, _):
            lv = lab_v[pl.ds(q * 16, 16)]
            for rr in range(16):
                base = lv[rr] * TCOLS
                for u in range(TCOLS // 16):
                    x = feat_v[q * 16 + rr, pl.ds(half + u * 16, 16)]
                    plsc.addupdate(acc_v.at[pl.ds(base + u * 16, 16)], x)
            return 0

        lax.fori_loop(0, RA // 16, grp, 0)
        return 0

    lax.fori_loop(0, N // RA, chunk, 0)
    pltpu.sync_copy(acc_v, out_hbm.at[wid])


def _means_body(lab_ref, sums_ref, means_ref):
    i = pl.program_id(0)
    lab = lab_ref[...]
    for t in range(CB):
        cid = i * CB + t
        cnt = jnp.sum(jnp.where(lab == cid, 1, 0))
        denom = jnp.maximum(cnt, 1).astype(jnp.float32)
        means_ref[pl.ds(t, 1), :] = sums_ref[pl.ds(t, 1), :] / denom


def _means_call(lab8, sums):
    return pl.pallas_call(
        _means_body,
        grid=(CPAD // CB,),
        in_specs=[
            pl.BlockSpec((8, N // 8), lambda i: (0, 0)),
            pl.BlockSpec((CB, D), lambda i: (i, 0)),
        ],
        out_specs=pl.BlockSpec((CB, D), lambda i: (i, 0)),
        out_shape=jax.ShapeDtypeStruct((CPAD, D), jnp.float32),
    )(lab8, sums)


def _dists_body(feat_hbm, lab_hbm, means_hbm, out_hbm, feat_v, mean_v, lab_v, ssq_v, sem):
    cidx = lax.axis_index("c")
    s = lax.axis_index("s")
    wid = s * NC + cidx

    lanes = lax.iota(jnp.int32, 16)

    def chunk(j, _):
        base = wid * RPW + j * RC
        pltpu.sync_copy(lab_hbm.at[pl.ds(base, RC)], lab_v)
        pltpu.sync_copy(feat_hbm.at[pl.ds(base, RC)], feat_v)
        pltpu.async_copy(means_hbm.at[lab_v], mean_v, sem).wait()

        def row(r, vec):
            def colstep(k2, acc):
                for u in range(4):
                    sl = pl.ds(k2 * 64 + u * 16, 16)
                    dd = feat_v[r, sl] - mean_v[r, sl]
                    acc = acc + dd * dd
                return acc

            acc = lax.fori_loop(0, D // 64, colstep,
                                jnp.zeros((16,), jnp.float32))
            return jnp.where(lanes == r, jnp.sum(acc), vec)

        vec = lax.fori_loop(0, RC, row, jnp.zeros((16,), jnp.float32))
        ssq_v[pl.ds(j * RC, RC)] = vec
        return 0

    lax.fori_loop(0, RPW // RC, chunk, 0)
    pltpu.sync_copy(ssq_v, out_hbm.at[pl.ds(wid * RPW, RPW)])


def _topk_body(lab_ref, ssq_ref, idx_ref, w_ref):
    i = pl.program_id(0)
    lab = lab_ref[...]
    d = jnp.sqrt(ssq_ref[...] + 1e-12)
    cls = i * CBL + lax.broadcasted_iota(jnp.int32, (CBL, 1), 0)
    member = lab == cls
    cnt = jnp.sum(member.astype(jnp.int32), axis=1, keepdims=True)
    cntf = jnp.maximum(cnt, 1).astype(jnp.float32)
    dist = jnp.where(member, d, jnp.inf)
    col = lax.broadcasted_iota(jnp.int32, (CBL, N), 1)
    col128 = lax.broadcasted_iota(jnp.int32, (CBL, 128), 1)
    idx_mat = jnp.zeros((CBL, 128), jnp.int32)
    w_mat = jnp.zeros((CBL, 128), jnp.float32)
    for k in range(K):
        m = jnp.min(dist, axis=1, keepdims=True)
        am = jnp.min(jnp.where(dist == m, col, N), axis=1, keepdims=True)
        wk = jnp.where(cnt >= K, jnp.float32(1.0 / K),
                       jnp.where(k < cnt, 1.0 / cntf, jnp.float32(0.0)))
        if k == 0:
            idx_mat = jnp.broadcast_to(am, (CBL, 128)).astype(jnp.int32)
        else:
            idx_mat = jnp.where(col128 == k, am, idx_mat)
        w_mat = jnp.where(col128 == k, wk, w_mat)
        dist = jnp.where(col == am, jnp.inf, dist)
    idx_ref[...] = idx_mat
    w_ref[...] = w_mat


def _topk_call(lab1, ssq1):
    return pl.pallas_call(
        _topk_body,
        grid=(CPAD // CBL,),
        in_specs=[
            pl.BlockSpec((1, N), lambda i: (0, 0)),
            pl.BlockSpec((1, N), lambda i: (0, 0)),
        ],
        out_specs=[
            pl.BlockSpec((CBL, 128), lambda i: (i, 0)),
            pl.BlockSpec((CBL, 128), lambda i: (i, 0)),
        ],
        out_shape=[
            jax.ShapeDtypeStruct((CPAD, 128), jnp.int32),
            jax.ShapeDtypeStruct((CPAD, 128), jnp.float32),
        ],
    )(lab1, ssq1)


def _protos_body(feat_hbm, idx_hbm, w_hbm, out_hbm,
                 idx_v, w_v, rows_v, proto8_v, sem):
    cidx = lax.axis_index("c")
    s = lax.axis_index("s")
    wid = s * NC + cidx
    base = wid * CPT
    pltpu.sync_copy(idx_hbm.at[pl.ds(base, CPT)], idx_v)
    pltpu.sync_copy(w_hbm.at[pl.ds(base, CPT)], w_v)

    def grp_step(g, _):
        def cls_step(t, _):
            tl = g * 8 + t
            pltpu.async_copy(
                feat_hbm.at[idx_v.at[tl, pl.ds(0, 8)]], rows_v, sem).wait()
            wvec = w_v[tl, pl.ds(0, 16)]
            w0 = wvec[0]
            w1 = wvec[1]
            w2 = wvec[2]
            w3 = wvec[3]
            w4 = wvec[4]

            def col(kk, _):
                sl = pl.ds(kk * 16, 16)
                acc = rows_v[0, sl] * w0
                acc = acc + rows_v[1, sl] * w1
                acc = acc + rows_v[2, sl] * w2
                acc = acc + rows_v[3, sl] * w3
                acc = acc + rows_v[4, sl] * w4
                proto8_v[t, sl] = acc
                return 0

            lax.fori_loop(0, D // 16, col, 0)
            return 0

        lax.fori_loop(0, 8, cls_step, 0)
        pltpu.sync_copy(proto8_v, out_hbm.at[pl.ds(base + g * 8, 8)])
        return 0

    lax.fori_loop(0, CPT // 8, grp_step, 0)


@functools.lru_cache(maxsize=1)
def _build_sc_kernels():
    mesh = plsc.VectorSubcoreMesh(
        core_axis_name="c", subcore_axis_name="s",
        num_cores=NC, num_subcores=NS)
    params = pltpu.CompilerParams(needs_layout_passes=False)
    segment_sums = pl.kernel(
        _segment_sums_body,
        out_type=jax.ShapeDtypeStruct((NW, CPAD * TCOLS), jnp.float32),
        mesh=mesh,
        compiler_params=params,
        scratch_types=[
            pltpu.VMEM((RA, 128), jnp.float32),
            pltpu.VMEM((RA,), jnp.int32),
            pltpu.VMEM((CPAD * TCOLS,), jnp.float32),
        ],
    )
    dists = pl.kernel(
        _dists_body,
        out_type=jax.ShapeDtypeStruct((N,), jnp.float32),
        mesh=mesh,
        compiler_params=params,
        scratch_types=[
            pltpu.VMEM((RC, D), jnp.float32),
            pltpu.VMEM((RC, D), jnp.float32),
            pltpu.VMEM((RC,), jnp.int32),
            pltpu.VMEM((RPW,), jnp.float32),
            pltpu.SemaphoreType.DMA,
        ],
    )
    protos_k = pl.kernel(
        _protos_body,
        out_type=jax.ShapeDtypeStruct((CPAD, D), jnp.float32),
        mesh=mesh,
        compiler_params=params,
        scratch_types=[
            pltpu.VMEM((CPT, 128), jnp.int32),
            pltpu.VMEM((CPT, 128), jnp.float32),
            pltpu.VMEM((8, D), jnp.float32),
            pltpu.VMEM((8, D), jnp.float32),
            pltpu.SemaphoreType.DMA,
        ],
    )
    return segment_sums, dists, protos_k


def kernel(features, labels):
    segment_sums, dists, protos_k = _build_sc_kernels()
    labels = labels.astype(jnp.int32)
    lab8 = labels.reshape(8, N // 8)
    sums3 = segment_sums(features, labels)
    sums = sums3.reshape(NW, CPAD, TCOLS).transpose(1, 0, 2).reshape(CPAD, D)
    means = _means_call(lab8, sums)
    ssq = dists(features, labels, means)
    idx128, w128 = _topk_call(labels.reshape(1, N), ssq.reshape(1, N))
    protos = protos_k(features, idx128, w128)
    return protos[:C]


# speedup vs baseline: 4.3528x; 1.2476x over previous
import functools

import jax
import jax.numpy as jnp
from jax import lax
from jax.experimental import pallas as pl
from jax.experimental.pallas import tpu as pltpu
from jax.experimental.pallas import tpu_sc as plsc

C = 1074
K = 5
D = 2048
N = 16384

NC, NS = 2, 16
NW = NC * NS
CPAD = 1280
CPT = CPAD // NW
SPC = CPAD // NS
DH = D // NC
RPS = N // NS
RA = 128
TCOLS = D // NW
RPW = N // NW
RC = 16
CB = 8
CBL = 128

def _segment_sums_body(feat_hbm, lab_hbm, out_hbm, feat_v, lab_v,
                       feat_v1, lab_v1, acc_v, sem_f0, sem_l0, sem_f1, sem_l1):
    cidx = lax.axis_index("c")
    s = lax.axis_index("s")
    wid = s * NC + cidx
    half = (wid % 2) * TCOLS

    def z(i, _):
        acc_v[pl.ds(i * 16, 16)] = jnp.zeros((16,), jnp.float32)
        return 0

    lax.fori_loop(0, CPAD * TCOLS // 16, z, 0)

    slab = (wid // 2) * 128

    def start(jc, fv, lv, fs, ls):
        rbase = jc * RA
        pltpu.async_copy(feat_hbm.at[pl.ds(rbase, RA), pl.ds(slab, 128)], fv, fs)
        pltpu.async_copy(lab_hbm.at[pl.ds(rbase, RA)], lv, ls)

    def wait(fv, lv, fs, ls):
        pltpu.make_async_copy(feat_hbm.at[pl.ds(0, RA), pl.ds(slab, 128)], fv, fs).wait()
        pltpu.make_async_copy(lab_hbm.at[pl.ds(0, RA)], lv, ls).wait()

    def process(fv, lv):
        def grp(q, _):
            lvec = lv[pl.ds(q * 16, 16)]
            bases = [lvec[rr] * TCOLS for rr in range(16)]
            for rr in range(16):
                for u in range(TCOLS // 16):
                    x = fv[q * 16 + rr, pl.ds(half + u * 16, 16)]
                    plsc.addupdate(acc_v.at[pl.ds(bases[rr] + u * 16, 16)], x)
            return 0

        lax.fori_loop(0, RA // 16, grp, 0)

    npairs = N // RA // 2
    start(0, feat_v, lab_v, sem_f0, sem_l0)

    def pairstep(p, _):
        jc0 = p * 2
        wait(feat_v, lab_v, sem_f0, sem_l0)
        start(jc0 + 1, feat_v1, lab_v1, sem_f1, sem_l1)
        process(feat_v, lab_v)
        wait(feat_v1, lab_v1, sem_f1, sem_l1)

        @pl.when(p < npairs - 1)
        def _():
            start(jc0 + 2, feat_v, lab_v, sem_f0, sem_l0)

        process(feat_v1, lab_v1)
        return 0

    lax.fori_loop(0, npairs, pairstep, 0)
    pltpu.sync_copy(acc_v, out_hbm.at[wid])


def _means_body(lab_ref, sums_ref, means_ref):
    i = pl.program_id(0)
    lab = lab_ref[...]
    for t in range(CB):
        cid = i * CB + t
        cnt = jnp.sum(jnp.where(lab == cid, 1, 0))
        denom = jnp.maximum(cnt, 1).astype(jnp.float32)
        means_ref[pl.ds(t, 1), :] = sums_ref[pl.ds(t, 1), :] / denom


def _means_call(lab8, sums):
    return pl.pallas_call(
        _means_body,
        grid=(CPAD // CB,),
        in_specs=[
            pl.BlockSpec((8, N // 8), lambda i: (0, 0)),
            pl.BlockSpec((CB, D), lambda i: (i, 0)),
        ],
        out_specs=pl.BlockSpec((CB, D), lambda i: (i, 0)),
        out_shape=jax.ShapeDtypeStruct((CPAD, D), jnp.float32),
    )(lab8, sums)


def _dists_body(feat_hbm, lab_hbm, means_hbm, out_hbm, feat_v, mean_v, lab_v, ssq_v, sem):
    cidx = lax.axis_index("c")
    s = lax.axis_index("s")
    wid = s * NC + cidx

    lanes = lax.iota(jnp.int32, 16)

    def chunk(j, _):
        base = wid * RPW + j * RC
        pltpu.sync_copy(lab_hbm.at[pl.ds(base, RC)], lab_v)
        pltpu.sync_copy(feat_hbm.at[pl.ds(base, RC)], feat_v)
        pltpu.async_copy(means_hbm.at[lab_v], mean_v, sem).wait()

        def row(r, vec):
            def colstep(k2, acc):
                for u in range(4):
                    sl = pl.ds(k2 * 64 + u * 16, 16)
                    dd = feat_v[r, sl] - mean_v[r, sl]
                    acc = acc + dd * dd
                return acc

            acc = lax.fori_loop(0, D // 64, colstep,
                                jnp.zeros((16,), jnp.float32))
            return jnp.where(lanes == r, jnp.sum(acc), vec)

        vec = lax.fori_loop(0, RC, row, jnp.zeros((16,), jnp.float32))
        ssq_v[pl.ds(j * RC, RC)] = vec
        return 0

    lax.fori_loop(0, RPW // RC, chunk, 0)
    pltpu.sync_copy(ssq_v, out_hbm.at[pl.ds(wid * RPW, RPW)])


def _topk_body(lab_ref, ssq_ref, idx_ref, w_ref):
    i = pl.program_id(0)
    lab = lab_ref[...]
    d = jnp.sqrt(ssq_ref[...] + 1e-12)
    cls = i * CBL + lax.broadcasted_iota(jnp.int32, (CBL, 1), 0)
    member = lab == cls
    cnt = jnp.sum(member.astype(jnp.int32), axis=1, keepdims=True)
    cntf = jnp.maximum(cnt, 1).astype(jnp.float32)
    dist = jnp.where(member, d, jnp.inf)
    col = lax.broadcasted_iota(jnp.int32, (CBL, N), 1)
    col128 = lax.broadcasted_iota(jnp.int32, (CBL, 128), 1)
    idx_mat = jnp.zeros((CBL, 128), jnp.int32)
    w_mat = jnp.zeros((CBL, 128), jnp.float32)
    for k in range(K):
        m = jnp.min(dist, axis=1, keepdims=True)
        am = jnp.min(jnp.where(dist == m, col, N), axis=1, keepdims=True)
        wk = jnp.where(cnt >= K, jnp.float32(1.0 / K),
                       jnp.where(k < cnt, 1.0 / cntf, jnp.float32(0.0)))
        if k == 0:
            idx_mat = jnp.broadcast_to(am, (CBL, 128)).astype(jnp.int32)
        else:
            idx_mat = jnp.where(col128 == k, am, idx_mat)
        w_mat = jnp.where(col128 == k, wk, w_mat)
        dist = jnp.where(col == am, jnp.inf, dist)
    idx_ref[...] = idx_mat
    w_ref[...] = w_mat


def _topk_call(lab1, ssq1):
    return pl.pallas_call(
        _topk_body,
        grid=(CPAD // CBL,),
        in_specs=[
            pl.BlockSpec((1, N), lambda i: (0, 0)),
            pl.BlockSpec((1, N), lambda i: (0, 0)),
        ],
        out_specs=[
            pl.BlockSpec((CBL, 128), lambda i: (i, 0)),
            pl.BlockSpec((CBL, 128), lambda i: (i, 0)),
        ],
        out_shape=[
            jax.ShapeDtypeStruct((CPAD, 128), jnp.int32),
            jax.ShapeDtypeStruct((CPAD, 128), jnp.float32),
        ],
    )(lab1, ssq1)


def _protos_body(feat_hbm, idx_hbm, w_hbm, out_hbm,
                 idx_v, w_v, rows_v, proto8_v, sem):
    cidx = lax.axis_index("c")
    s = lax.axis_index("s")
    wid = s * NC + cidx
    base = wid * CPT
    pltpu.sync_copy(idx_hbm.at[pl.ds(base, CPT)], idx_v)
    pltpu.sync_copy(w_hbm.at[pl.ds(base, CPT)], w_v)

    def grp_step(g, _):
        def cls_step(t, _):
            tl = g * 8 + t
            pltpu.async_copy(
                feat_hbm.at[idx_v.at[tl, pl.ds(0, 8)]], rows_v, sem).wait()
            wvec = w_v[tl, pl.ds(0, 16)]
            w0 = wvec[0]
            w1 = wvec[1]
            w2 = wvec[2]
            w3 = wvec[3]
            w4 = wvec[4]

            def col(kk, _):
                sl = pl.ds(kk * 16, 16)
                acc = rows_v[0, sl] * w0
                acc = acc + rows_v[1, sl] * w1
                acc = acc + rows_v[2, sl] * w2
                acc = acc + rows_v[3, sl] * w3
                acc = acc + rows_v[4, sl] * w4
                proto8_v[t, sl] = acc
                return 0

            lax.fori_loop(0, D // 16, col, 0)
            return 0

        lax.fori_loop(0, 8, cls_step, 0)
        pltpu.sync_copy(proto8_v, out_hbm.at[pl.ds(base + g * 8, 8)])
        return 0

    lax.fori_loop(0, CPT // 8, grp_step, 0)


@functools.lru_cache(maxsize=1)
def _build_sc_kernels():
    mesh = plsc.VectorSubcoreMesh(
        core_axis_name="c", subcore_axis_name="s",
        num_cores=NC, num_subcores=NS)
    params = pltpu.CompilerParams(needs_layout_passes=False)
    segment_sums = pl.kernel(
        _segment_sums_body,
        out_type=jax.ShapeDtypeStruct((NW, CPAD * TCOLS), jnp.float32),
        mesh=mesh,
        compiler_params=params,
        scratch_types=[
            pltpu.VMEM((RA, 128), jnp.float32),
            pltpu.VMEM((RA,), jnp.int32),
            pltpu.VMEM((RA, 128), jnp.float32),
            pltpu.VMEM((RA,), jnp.int32),
            pltpu.VMEM((CPAD * TCOLS,), jnp.float32),
            pltpu.SemaphoreType.DMA,
            pltpu.SemaphoreType.DMA,
            pltpu.SemaphoreType.DMA,
            pltpu.SemaphoreType.DMA,
        ],
    )
    dists = pl.kernel(
        _dists_body,
        out_type=jax.ShapeDtypeStruct((N,), jnp.float32),
        mesh=mesh,
        compiler_params=params,
        scratch_types=[
            pltpu.VMEM((RC, D), jnp.float32),
            pltpu.VMEM((RC, D), jnp.float32),
            pltpu.VMEM((RC,), jnp.int32),
            pltpu.VMEM((RPW,), jnp.float32),
            pltpu.SemaphoreType.DMA,
        ],
    )
    protos_k = pl.kernel(
        _protos_body,
        out_type=jax.ShapeDtypeStruct((CPAD, D), jnp.float32),
        mesh=mesh,
        compiler_params=params,
        scratch_types=[
            pltpu.VMEM((CPT, 128), jnp.int32),
            pltpu.VMEM((CPT, 128), jnp.float32),
            pltpu.VMEM((8, D), jnp.float32),
            pltpu.VMEM((8, D), jnp.float32),
            pltpu.SemaphoreType.DMA,
        ],
    )
    return segment_sums, dists, protos_k


def kernel(features, labels):
    segment_sums, dists, protos_k = _build_sc_kernels()
    labels = labels.astype(jnp.int32)
    lab8 = labels.reshape(8, N // 8)
    sums3 = segment_sums(features, labels)
    sums = sums3.reshape(NW, CPAD, TCOLS).transpose(1, 0, 2).reshape(CPAD, D)
    means = _means_call(lab8, sums)
    ssq = dists(features, labels, means)
    idx128, w128 = _topk_call(labels.reshape(1, N), ssq.reshape(1, N))
    protos = protos_k(features, idx128, w128)
    return protos[:C]


# speedup vs baseline: 4.6198x; 1.0613x over previous
import functools

import jax
import jax.numpy as jnp
from jax import lax
from jax.experimental import pallas as pl
from jax.experimental.pallas import tpu as pltpu
from jax.experimental.pallas import tpu_sc as plsc

C = 1074
K = 5
D = 2048
N = 16384

NC, NS = 2, 16
NW = NC * NS
CPAD = 1280
CPT = CPAD // NW
SPC = CPAD // NS
DH = D // NC
RPS = N // NS
RA = 128
TCOLS = D // NW
RPW = N // NW
RC = 16
CB = 8
CBL = 128

def _segment_sums_body(feat_hbm, lab_hbm, out_hbm, feat_v, lab_v,
                       feat_v1, lab_v1, acc_v, sem_f0, sem_l0, sem_f1, sem_l1):
    cidx = lax.axis_index("c")
    s = lax.axis_index("s")
    wid = s * NC + cidx
    half = (wid % 2) * TCOLS

    def z(i, _):
        acc_v[pl.ds(i * 16, 16)] = jnp.zeros((16,), jnp.float32)
        return 0

    lax.fori_loop(0, CPAD * TCOLS // 16, z, 0)

    slab = (wid // 2) * 128

    def start(jc, fv, lv, fs, ls):
        rbase = jc * RA
        pltpu.async_copy(feat_hbm.at[pl.ds(rbase, RA), pl.ds(slab, 128)], fv, fs)
        pltpu.async_copy(lab_hbm.at[pl.ds(rbase, RA)], lv, ls)

    def wait(fv, lv, fs, ls):
        pltpu.make_async_copy(feat_hbm.at[pl.ds(0, RA), pl.ds(slab, 128)], fv, fs).wait()
        pltpu.make_async_copy(lab_hbm.at[pl.ds(0, RA)], lv, ls).wait()

    def process(fv, lv):
        def grp(q, _):
            lvec = lv[pl.ds(q * 16, 16)]
            bases = [lvec[rr] * TCOLS for rr in range(16)]
            for rr in range(16):
                for u in range(TCOLS // 16):
                    x = fv[q * 16 + rr, pl.ds(half + u * 16, 16)]
                    plsc.addupdate(acc_v.at[pl.ds(bases[rr] + u * 16, 16)], x)
            return 0

        lax.fori_loop(0, RA // 16, grp, 0)

    npairs = N // RA // 2
    start(0, feat_v, lab_v, sem_f0, sem_l0)

    def pairstep(p, _):
        jc0 = p * 2
        wait(feat_v, lab_v, sem_f0, sem_l0)
        start(jc0 + 1, feat_v1, lab_v1, sem_f1, sem_l1)
        process(feat_v, lab_v)
        wait(feat_v1, lab_v1, sem_f1, sem_l1)

        @pl.when(p < npairs - 1)
        def _():
            start(jc0 + 2, feat_v, lab_v, sem_f0, sem_l0)

        process(feat_v1, lab_v1)
        return 0

    lax.fori_loop(0, npairs, pairstep, 0)
    pltpu.sync_copy(acc_v, out_hbm.at[wid])


def _means_body(lab_ref, sums_ref, means_ref):
    i = pl.program_id(0)
    lab = lab_ref[...]
    for t in range(CB):
        cid = i * CB + t
        cnt = jnp.sum(jnp.where(lab == cid, 1, 0))
        denom = jnp.maximum(cnt, 1).astype(jnp.float32)
        means_ref[pl.ds(t, 1), :] = sums_ref[pl.ds(t, 1), :] / denom


def _means_call(lab8, sums):
    return pl.pallas_call(
        _means_body,
        grid=(CPAD // CB,),
        in_specs=[
            pl.BlockSpec((8, N // 8), lambda i: (0, 0)),
            pl.BlockSpec((CB, D), lambda i: (i, 0)),
        ],
        out_specs=pl.BlockSpec((CB, D), lambda i: (i, 0)),
        out_shape=jax.ShapeDtypeStruct((CPAD, D), jnp.float32),
    )(lab8, sums)


def _dists_body(feat_hbm, lab_hbm, means_hbm, out_hbm,
                feat0, feat1, lab0, lab1, mean_v, ssq_v,
                sf0, sf1, sl0, sl1, sg):
    cidx = lax.axis_index("c")
    s = lax.axis_index("s")
    wid = s * NC + cidx

    lanes = lax.iota(jnp.int32, 16)

    def startf(j, fv, lv, fs, ls):
        base = wid * RPW + j * RC
        pltpu.async_copy(feat_hbm.at[pl.ds(base, RC)], fv, fs)
        pltpu.async_copy(lab_hbm.at[pl.ds(base, RC)], lv, ls)

    def waitf(fv, lv, fs, ls):
        pltpu.make_async_copy(feat_hbm.at[pl.ds(0, RC)], fv, fs).wait()
        pltpu.make_async_copy(lab_hbm.at[pl.ds(0, RC)], lv, ls).wait()

    def process(j, fv, lv):
        pltpu.async_copy(means_hbm.at[lv], mean_v, sg).wait()

        def row(r, vec):
            def colstep(k2, acc):
                for u in range(4):
                    sl = pl.ds(k2 * 64 + u * 16, 16)
                    dd = fv[r, sl] - mean_v[r, sl]
                    acc = acc + dd * dd
                return acc

            acc = lax.fori_loop(0, D // 64, colstep,
                                jnp.zeros((16,), jnp.float32))
            return jnp.where(lanes == r, jnp.sum(acc), vec)

        vec = lax.fori_loop(0, RC, row, jnp.zeros((16,), jnp.float32))
        ssq_v[pl.ds(j * RC, RC)] = vec

    npairs = RPW // RC // 2
    startf(0, feat0, lab0, sf0, sl0)

    def pairstep(p, _):
        j0 = 2 * p
        waitf(feat0, lab0, sf0, sl0)
        startf(j0 + 1, feat1, lab1, sf1, sl1)
        process(j0, feat0, lab0)
        waitf(feat1, lab1, sf1, sl1)

        @pl.when(p < npairs - 1)
        def _():
            startf(j0 + 2, feat0, lab0, sf0, sl0)

        process(j0 + 1, feat1, lab1)
        return 0

    lax.fori_loop(0, npairs, pairstep, 0)
    pltpu.sync_copy(ssq_v, out_hbm.at[pl.ds(wid * RPW, RPW)])


def _topk_body(lab_ref, ssq_ref, idx_ref, w_ref):
    i = pl.program_id(0)
    lab = lab_ref[...]
    d = jnp.sqrt(ssq_ref[...] + 1e-12)
    cls = i * CBL + lax.broadcasted_iota(jnp.int32, (CBL, 1), 0)
    member = lab == cls
    cnt = jnp.sum(member.astype(jnp.int32), axis=1, keepdims=True)
    cntf = jnp.maximum(cnt, 1).astype(jnp.float32)
    dist = jnp.where(member, d, jnp.inf)
    col = lax.broadcasted_iota(jnp.int32, (CBL, N), 1)
    col128 = lax.broadcasted_iota(jnp.int32, (CBL, 128), 1)
    idx_mat = jnp.zeros((CBL, 128), jnp.int32)
    w_mat = jnp.zeros((CBL, 128), jnp.float32)
    for k in range(K):
        m = jnp.min(dist, axis=1, keepdims=True)
        am = jnp.min(jnp.where(dist == m, col, N), axis=1, keepdims=True)
        wk = jnp.where(cnt >= K, jnp.float32(1.0 / K),
                       jnp.where(k < cnt, 1.0 / cntf, jnp.float32(0.0)))
        if k == 0:
            idx_mat = jnp.broadcast_to(am, (CBL, 128)).astype(jnp.int32)
        else:
            idx_mat = jnp.where(col128 == k, am, idx_mat)
        w_mat = jnp.where(col128 == k, wk, w_mat)
        dist = jnp.where(col == am, jnp.inf, dist)
    idx_ref[...] = idx_mat
    w_ref[...] = w_mat


def _topk_call(lab1, ssq1):
    return pl.pallas_call(
        _topk_body,
        grid=(CPAD // CBL,),
        in_specs=[
            pl.BlockSpec((1, N), lambda i: (0, 0)),
            pl.BlockSpec((1, N), lambda i: (0, 0)),
        ],
        out_specs=[
            pl.BlockSpec((CBL, 128), lambda i: (i, 0)),
            pl.BlockSpec((CBL, 128), lambda i: (i, 0)),
        ],
        out_shape=[
            jax.ShapeDtypeStruct((CPAD, 128), jnp.int32),
            jax.ShapeDtypeStruct((CPAD, 128), jnp.float32),
        ],
    )(lab1, ssq1)


def _protos_body(feat_hbm, idx_hbm, w_hbm, out_hbm,
                 idx_v, w_v, rows0, rows1, proto_v, sem0, sem1):
    cidx = lax.axis_index("c")
    s = lax.axis_index("s")
    wid = s * NC + cidx
    base = wid * CPT
    pltpu.sync_copy(idx_hbm.at[pl.ds(base, CPT)], idx_v)
    pltpu.sync_copy(w_hbm.at[pl.ds(base, CPT)], w_v)

    def startg(t, rv, sem):
        pltpu.async_copy(feat_hbm.at[idx_v.at[t, pl.ds(0, 8)]], rv, sem)

    def waitg(rv, sem):
        pltpu.make_async_copy(feat_hbm.at[pl.ds(0, 8)], rv, sem).wait()

    def compute(t, rv):
        wvec = w_v[t, pl.ds(0, 16)]
        w0 = wvec[0]
        w1 = wvec[1]
        w2 = wvec[2]
        w3 = wvec[3]
        w4 = wvec[4]

        def col(kk, _):
            sl = pl.ds(kk * 16, 16)
            acc = rv[0, sl] * w0
            acc = acc + rv[1, sl] * w1
            acc = acc + rv[2, sl] * w2
            acc = acc + rv[3, sl] * w3
            acc = acc + rv[4, sl] * w4
            proto_v[t, sl] = acc
            return 0

        lax.fori_loop(0, D // 16, col, 0)

    npairs = CPT // 2
    startg(0, rows0, sem0)

    def pairstep(p, _):
        t0 = 2 * p
        waitg(rows0, sem0)
        startg(t0 + 1, rows1, sem1)
        compute(t0, rows0)
        waitg(rows1, sem1)

        @pl.when(p < npairs - 1)
        def _():
            startg(t0 + 2, rows0, sem0)

        compute(t0 + 1, rows1)
        return 0

    lax.fori_loop(0, npairs, pairstep, 0)
    pltpu.sync_copy(proto_v, out_hbm.at[pl.ds(base, CPT)])


@functools.lru_cache(maxsize=1)
def _build_sc_kernels():
    mesh = plsc.VectorSubcoreMesh(
        core_axis_name="c", subcore_axis_name="s",
        num_cores=NC, num_subcores=NS)
    params = pltpu.CompilerParams(needs_layout_passes=False)
    segment_sums = pl.kernel(
        _segment_sums_body,
        out_type=jax.ShapeDtypeStruct((NW, CPAD * TCOLS), jnp.float32),
        mesh=mesh,
        compiler_params=params,
        scratch_types=[
            pltpu.VMEM((RA, 128), jnp.float32),
            pltpu.VMEM((RA,), jnp.int32),
            pltpu.VMEM((RA, 128), jnp.float32),
            pltpu.VMEM((RA,), jnp.int32),
            pltpu.VMEM((CPAD * TCOLS,), jnp.float32),
            pltpu.SemaphoreType.DMA,
            pltpu.SemaphoreType.DMA,
            pltpu.SemaphoreType.DMA,
            pltpu.SemaphoreType.DMA,
        ],
    )
    dists = pl.kernel(
        _dists_body,
        out_type=jax.ShapeDtypeStruct((N,), jnp.float32),
        mesh=mesh,
        compiler_params=params,
        scratch_types=[
            pltpu.VMEM((RC, D), jnp.float32),
            pltpu.VMEM((RC, D), jnp.float32),
            pltpu.VMEM((RC,), jnp.int32),
            pltpu.VMEM((RC,), jnp.int32),
            pltpu.VMEM((RC, D), jnp.float32),
            pltpu.VMEM((RPW,), jnp.float32),
            pltpu.SemaphoreType.DMA,
            pltpu.SemaphoreType.DMA,
            pltpu.SemaphoreType.DMA,
            pltpu.SemaphoreType.DMA,
            pltpu.SemaphoreType.DMA,
        ],
    )
    protos_k = pl.kernel(
        _protos_body,
        out_type=jax.ShapeDtypeStruct((CPAD, D), jnp.float32),
        mesh=mesh,
        compiler_params=params,
        scratch_types=[
            pltpu.VMEM((CPT, 128), jnp.int32),
            pltpu.VMEM((CPT, 128), jnp.float32),
            pltpu.VMEM((8, D), jnp.float32),
            pltpu.VMEM((8, D), jnp.float32),
            pltpu.VMEM((CPT, D), jnp.float32),
            pltpu.SemaphoreType.DMA,
            pltpu.SemaphoreType.DMA,
        ],
    )
    return segment_sums, dists, protos_k


def kernel(features, labels):
    segment_sums, dists, protos_k = _build_sc_kernels()
    labels = labels.astype(jnp.int32)
    lab8 = labels.reshape(8, N // 8)
    sums3 = segment_sums(features, labels)
    sums = sums3.reshape(NW, CPAD, TCOLS).transpose(1, 0, 2).reshape(CPAD, D)
    means = _means_call(lab8, sums)
    ssq = dists(features, labels, means)
    idx128, w128 = _topk_call(labels.reshape(1, N), ssq.reshape(1, N))
    protos = protos_k(features, idx128, w128)
    return protos[:C]


# speedup vs baseline: 4.8235x; 1.0441x over previous
import functools

import jax
import jax.numpy as jnp
from jax import lax
from jax.experimental import pallas as pl
from jax.experimental.pallas import tpu as pltpu
from jax.experimental.pallas import tpu_sc as plsc

C = 1074
K = 5
D = 2048
N = 16384

NC, NS = 2, 16
NW = NC * NS
CPAD = 1280
CPT = CPAD // NW
SPC = CPAD // NS
DH = D // NC
RPS = N // NS
RA = 128
TCOLS = D // NW
RPW = N // NW
RC = 16
CB = 8
CBL = 128

def _segment_sums_body(feat_hbm, lab_hbm, out_hbm, feat_v, lab_v,
                       feat_v1, lab_v1, acc_v, sem_f0, sem_l0, sem_f1, sem_l1):
    cidx = lax.axis_index("c")
    s = lax.axis_index("s")
    wid = s * NC + cidx
    half = (wid % 2) * TCOLS

    def z(i, _):
        acc_v[pl.ds(i * 16, 16)] = jnp.zeros((16,), jnp.float32)
        return 0

    lax.fori_loop(0, CPAD * TCOLS // 16, z, 0)

    slab = (wid // 2) * 128

    def start(jc, fv, lv, fs, ls):
        rbase = jc * RA
        pltpu.async_copy(feat_hbm.at[pl.ds(rbase, RA), pl.ds(slab, 128)], fv, fs)
        pltpu.async_copy(lab_hbm.at[pl.ds(rbase, RA)], lv, ls)

    def wait(fv, lv, fs, ls):
        pltpu.make_async_copy(feat_hbm.at[pl.ds(0, RA), pl.ds(slab, 128)], fv, fs).wait()
        pltpu.make_async_copy(lab_hbm.at[pl.ds(0, RA)], lv, ls).wait()

    def process(fv, lv):
        def grp(q, _):
            lvec = lv[pl.ds(q * 16, 16)]
            bases = [lvec[rr] * TCOLS for rr in range(16)]
            for rr in range(16):
                for u in range(TCOLS // 16):
                    x = fv[q * 16 + rr, pl.ds(half + u * 16, 16)]
                    plsc.addupdate(acc_v.at[pl.ds(bases[rr] + u * 16, 16)], x)
            return 0

        lax.fori_loop(0, RA // 16, grp, 0)

    npairs = N // RA // 2
    start(0, feat_v, lab_v, sem_f0, sem_l0)

    def pairstep(p, _):
        jc0 = p * 2
        wait(feat_v, lab_v, sem_f0, sem_l0)
        start(jc0 + 1, feat_v1, lab_v1, sem_f1, sem_l1)
        process(feat_v, lab_v)
        wait(feat_v1, lab_v1, sem_f1, sem_l1)

        @pl.when(p < npairs - 1)
        def _():
            start(jc0 + 2, feat_v, lab_v, sem_f0, sem_l0)

        process(feat_v1, lab_v1)
        return 0

    lax.fori_loop(0, npairs, pairstep, 0)
    pltpu.sync_copy(acc_v, out_hbm.at[wid])


def _means_body(lab_ref, sums_ref, means_ref):
    i = pl.program_id(0)
    lab = lab_ref[...]
    for t in range(CB):
        cid = i * CB + t
        cnt = jnp.sum(jnp.where(lab == cid, 1, 0))
        denom = jnp.maximum(cnt, 1).astype(jnp.float32)
        means_ref[pl.ds(t, 1), :] = sums_ref[pl.ds(t, 1), :] / denom


def _means_call(lab8, sums):
    return pl.pallas_call(
        _means_body,
        grid=(CPAD // CB,),
        in_specs=[
            pl.BlockSpec((8, N // 8), lambda i: (0, 0)),
            pl.BlockSpec((CB, D), lambda i: (i, 0)),
        ],
        out_specs=pl.BlockSpec((CB, D), lambda i: (i, 0)),
        out_shape=jax.ShapeDtypeStruct((CPAD, D), jnp.float32),
    )(lab8, sums)


def _dists_body(feat_hbm, lab_hbm, means_hbm, out_hbm,
                feat0, feat1, lab0, lab1, mean_v, ssq_v,
                sf0, sf1, sl0, sl1, sg):
    cidx = lax.axis_index("c")
    s = lax.axis_index("s")
    wid = s * NC + cidx

    lanes = lax.iota(jnp.int32, 16)

    def startf(j, fv, lv, fs, ls):
        base = wid * RPW + j * RC
        pltpu.async_copy(feat_hbm.at[pl.ds(base, RC)], fv, fs)
        pltpu.async_copy(lab_hbm.at[pl.ds(base, RC)], lv, ls)

    def waitf(fv, lv, fs, ls):
        pltpu.make_async_copy(feat_hbm.at[pl.ds(0, RC)], fv, fs).wait()
        pltpu.make_async_copy(lab_hbm.at[pl.ds(0, RC)], lv, ls).wait()

    def process(j, fv, lv):
        pltpu.async_copy(means_hbm.at[lv], mean_v, sg).wait()

        def row(r, vec):
            def colstep(k2, acc):
                for u in range(4):
                    sl = pl.ds(k2 * 64 + u * 16, 16)
                    dd = fv[r, sl] - mean_v[r, sl]
                    acc = acc + dd * dd
                return acc

            acc = lax.fori_loop(0, D // 64, colstep,
                                jnp.zeros((16,), jnp.float32))
            return jnp.where(lanes == r, jnp.sum(acc), vec)

        vec = lax.fori_loop(0, RC, row, jnp.zeros((16,), jnp.float32))
        ssq_v[pl.ds(j * RC, RC)] = vec

    npairs = RPW // RC // 2
    startf(0, feat0, lab0, sf0, sl0)

    def pairstep(p, _):
        j0 = 2 * p
        waitf(feat0, lab0, sf0, sl0)
        startf(j0 + 1, feat1, lab1, sf1, sl1)
        process(j0, feat0, lab0)
        waitf(feat1, lab1, sf1, sl1)

        @pl.when(p < npairs - 1)
        def _():
            startf(j0 + 2, feat0, lab0, sf0, sl0)

        process(j0 + 1, feat1, lab1)
        return 0

    lax.fori_loop(0, npairs, pairstep, 0)
    pltpu.sync_copy(ssq_v, out_hbm.at[pl.ds(wid * RPW, RPW)])


def _topk_body(lab_ref, ssq_ref, idx_ref, w_ref):
    i = pl.program_id(0)
    lab = lab_ref[...]
    d = jnp.sqrt(ssq_ref[...] + 1e-12)
    cls = i * CBL + lax.broadcasted_iota(jnp.int32, (CBL, 1), 0)
    member = lab == cls
    cnt = jnp.sum(member.astype(jnp.int32), axis=1, keepdims=True)
    cntf = jnp.maximum(cnt, 1).astype(jnp.float32)
    dist = jnp.where(member, d, jnp.inf)
    col = lax.broadcasted_iota(jnp.int32, (CBL, N), 1)
    col128 = lax.broadcasted_iota(jnp.int32, (CBL, 128), 1)
    idx_mat = jnp.zeros((CBL, 128), jnp.int32)
    w_mat = jnp.zeros((CBL, 128), jnp.float32)
    for k in range(K):
        m = jnp.min(dist, axis=1, keepdims=True)
        am = jnp.min(jnp.where(dist == m, col, N), axis=1, keepdims=True)
        wk = jnp.where(cnt >= K, jnp.float32(1.0 / K),
                       jnp.where(k < cnt, 1.0 / cntf, jnp.float32(0.0)))
        if k == 0:
            idx_mat = jnp.broadcast_to(am, (CBL, 128)).astype(jnp.int32)
        else:
            idx_mat = jnp.where(col128 == k, am, idx_mat)
        w_mat = jnp.where(col128 == k, wk, w_mat)
        dist = jnp.where(col == am, jnp.inf, dist)
    idx_ref[...] = idx_mat
    w_ref[...] = w_mat


def _topk_call(lab1, ssq1):
    return pl.pallas_call(
        _topk_body,
        grid=(CPAD // CBL,),
        in_specs=[
            pl.BlockSpec((1, N), lambda i: (0, 0)),
            pl.BlockSpec((1, N), lambda i: (0, 0)),
        ],
        out_specs=[
            pl.BlockSpec((CBL, 128), lambda i: (i, 0)),
            pl.BlockSpec((CBL, 128), lambda i: (i, 0)),
        ],
        out_shape=[
            jax.ShapeDtypeStruct((CPAD, 128), jnp.int32),
            jax.ShapeDtypeStruct((CPAD, 128), jnp.float32),
        ],
    )(lab1, ssq1)


def _protos_body(feat_hbm, idx_hbm, w_hbm, out_hbm,
                 idx_v, w_v, rows0, rows1, proto_v, sem0, sem1):
    cidx = lax.axis_index("c")
    s = lax.axis_index("s")
    wid = s * NC + cidx
    base = wid * CPT
    pltpu.sync_copy(idx_hbm.at[pl.ds(base, CPT)], idx_v)
    pltpu.sync_copy(w_hbm.at[pl.ds(base, CPT)], w_v)

    def startg(t, rv, sem):
        pltpu.async_copy(feat_hbm.at[idx_v.at[t, pl.ds(0, K)]], rv, sem)

    def waitg(rv, sem):
        pltpu.make_async_copy(feat_hbm.at[pl.ds(0, K)], rv, sem).wait()

    def compute(t, rv):
        wvec = w_v[t, pl.ds(0, 16)]
        w0 = wvec[0]
        w1 = wvec[1]
        w2 = wvec[2]
        w3 = wvec[3]
        w4 = wvec[4]

        def col(kk, _):
            sl = pl.ds(kk * 16, 16)
            acc = rv[0, sl] * w0
            acc = acc + rv[1, sl] * w1
            acc = acc + rv[2, sl] * w2
            acc = acc + rv[3, sl] * w3
            acc = acc + rv[4, sl] * w4
            proto_v[t, sl] = acc
            return 0

        lax.fori_loop(0, D // 16, col, 0)

    npairs = CPT // 2
    startg(0, rows0, sem0)

    def pairstep(p, _):
        t0 = 2 * p
        waitg(rows0, sem0)
        startg(t0 + 1, rows1, sem1)
        compute(t0, rows0)
        waitg(rows1, sem1)

        @pl.when(p < npairs - 1)
        def _():
            startg(t0 + 2, rows0, sem0)

        compute(t0 + 1, rows1)
        return 0

    lax.fori_loop(0, npairs, pairstep, 0)
    pltpu.sync_copy(proto_v, out_hbm.at[pl.ds(base, CPT)])


@functools.lru_cache(maxsize=1)
def _build_sc_kernels():
    mesh = plsc.VectorSubcoreMesh(
        core_axis_name="c", subcore_axis_name="s",
        num_cores=NC, num_subcores=NS)
    params = pltpu.CompilerParams(needs_layout_passes=False)
    segment_sums = pl.kernel(
        _segment_sums_body,
        out_type=jax.ShapeDtypeStruct((NW, CPAD * TCOLS), jnp.float32),
        mesh=mesh,
        compiler_params=params,
        scratch_types=[
            pltpu.VMEM((RA, 128), jnp.float32),
            pltpu.VMEM((RA,), jnp.int32),
            pltpu.VMEM((RA, 128), jnp.float32),
            pltpu.VMEM((RA,), jnp.int32),
            pltpu.VMEM((CPAD * TCOLS,), jnp.float32),
            pltpu.SemaphoreType.DMA,
            pltpu.SemaphoreType.DMA,
            pltpu.SemaphoreType.DMA,
            pltpu.SemaphoreType.DMA,
        ],
    )
    dists = pl.kernel(
        _dists_body,
        out_type=jax.ShapeDtypeStruct((N,), jnp.float32),
        mesh=mesh,
        compiler_params=params,
        scratch_types=[
            pltpu.VMEM((RC, D), jnp.float32),
            pltpu.VMEM((RC, D), jnp.float32),
            pltpu.VMEM((RC,), jnp.int32),
            pltpu.VMEM((RC,), jnp.int32),
            pltpu.VMEM((RC, D), jnp.float32),
            pltpu.VMEM((RPW,), jnp.float32),
            pltpu.SemaphoreType.DMA,
            pltpu.SemaphoreType.DMA,
            pltpu.SemaphoreType.DMA,
            pltpu.SemaphoreType.DMA,
            pltpu.SemaphoreType.DMA,
        ],
    )
    protos_k = pl.kernel(
        _protos_body,
        out_type=jax.ShapeDtypeStruct((CPAD, D), jnp.float32),
        mesh=mesh,
        compiler_params=params,
        scratch_types=[
            pltpu.VMEM((CPT, 128), jnp.int32),
            pltpu.VMEM((CPT, 128), jnp.float32),
            pltpu.VMEM((K, D), jnp.float32),
            pltpu.VMEM((K, D), jnp.float32),
            pltpu.VMEM((CPT, D), jnp.float32),
            pltpu.SemaphoreType.DMA,
            pltpu.SemaphoreType.DMA,
        ],
    )
    return segment_sums, dists, protos_k


def kernel(features, labels):
    segment_sums, dists, protos_k = _build_sc_kernels()
    labels = labels.astype(jnp.int32)
    lab8 = labels.reshape(8, N // 8)
    sums3 = segment_sums(features, labels)
    sums = sums3.reshape(NW, CPAD, TCOLS).transpose(1, 0, 2).reshape(CPAD, D)
    means = _means_call(lab8, sums)
    ssq = dists(features, labels, means)
    idx128, w128 = _topk_call(labels.reshape(1, N), ssq.reshape(1, N))
    protos = protos_k(features, idx128, w128)
    return protos[:C]


# speedup vs baseline: 4.9843x; 1.0333x over previous
import functools

import jax
import jax.numpy as jnp
from jax import lax
from jax.experimental import pallas as pl
from jax.experimental.pallas import tpu as pltpu
from jax.experimental.pallas import tpu_sc as plsc

C = 1074
K = 5
D = 2048
N = 16384

NC, NS = 2, 16
NW = NC * NS
CPAD = 1280
CPT = CPAD // NW
SPC = CPAD // NS
DH = D // NC
RPS = N // NS
RA = 128
TCOLS = D // NW
RPW = N // NW
RC = 16
CB = 8
CBL = 128

def _segment_sums_body(feat_hbm, lab_hbm, out_hbm, feat_v, lab_v,
                       feat_v1, lab_v1, acc_v, sem_f0, sem_l0, sem_f1, sem_l1):
    cidx = lax.axis_index("c")
    s = lax.axis_index("s")
    wid = s * NC + cidx
    half = (wid % 2) * TCOLS

    def z(i, _):
        acc_v[pl.ds(i * 16, 16)] = jnp.zeros((16,), jnp.float32)
        return 0

    lax.fori_loop(0, CPAD * TCOLS // 16, z, 0)

    slab = (wid // 2) * 128

    def start(jc, fv, lv, fs, ls):
        rbase = jc * RA
        pltpu.async_copy(feat_hbm.at[pl.ds(rbase, RA), pl.ds(slab, 128)], fv, fs)
        pltpu.async_copy(lab_hbm.at[pl.ds(rbase, RA)], lv, ls)

    def wait(fv, lv, fs, ls):
        pltpu.make_async_copy(feat_hbm.at[pl.ds(0, RA), pl.ds(slab, 128)], fv, fs).wait()
        pltpu.make_async_copy(lab_hbm.at[pl.ds(0, RA)], lv, ls).wait()

    def process(fv, lv):
        @plsc.parallel_loop(0, RA // 16)
        def grp(q):
            lvec = lv[pl.ds(q * 16, 16)]
            bases = [lvec[rr] * TCOLS for rr in range(16)]
            for rr in range(16):
                for u in range(TCOLS // 16):
                    x = fv[q * 16 + rr, pl.ds(half + u * 16, 16)]
                    plsc.addupdate(acc_v.at[pl.ds(bases[rr] + u * 16, 16)], x)

    npairs = N // RA // 2
    start(0, feat_v, lab_v, sem_f0, sem_l0)

    def pairstep(p, _):
        jc0 = p * 2
        wait(feat_v, lab_v, sem_f0, sem_l0)
        start(jc0 + 1, feat_v1, lab_v1, sem_f1, sem_l1)
        process(feat_v, lab_v)
        wait(feat_v1, lab_v1, sem_f1, sem_l1)

        @pl.when(p < npairs - 1)
        def _():
            start(jc0 + 2, feat_v, lab_v, sem_f0, sem_l0)

        process(feat_v1, lab_v1)
        return 0

    lax.fori_loop(0, npairs, pairstep, 0)
    pltpu.sync_copy(acc_v, out_hbm.at[wid])


def _means_body(lab_ref, sums_ref, means_ref):
    i = pl.program_id(0)
    lab = lab_ref[...]
    for t in range(CB):
        cid = i * CB + t
        cnt = jnp.sum(jnp.where(lab == cid, 1, 0))
        denom = jnp.maximum(cnt, 1).astype(jnp.float32)
        means_ref[pl.ds(t, 1), :] = sums_ref[pl.ds(t, 1), :] / denom


def _means_call(lab8, sums):
    return pl.pallas_call(
        _means_body,
        grid=(CPAD // CB,),
        in_specs=[
            pl.BlockSpec((8, N // 8), lambda i: (0, 0)),
            pl.BlockSpec((CB, D), lambda i: (i, 0)),
        ],
        out_specs=pl.BlockSpec((CB, D), lambda i: (i, 0)),
        out_shape=jax.ShapeDtypeStruct((CPAD, D), jnp.float32),
    )(lab8, sums)


def _dists_body(feat_hbm, lab_hbm, means_hbm, out_hbm,
                feat0, feat1, lab0, lab1, mean_v, ssq_v,
                sf0, sf1, sl0, sl1, sg):
    cidx = lax.axis_index("c")
    s = lax.axis_index("s")
    wid = s * NC + cidx

    lanes = lax.iota(jnp.int32, 16)

    def startf(j, fv, lv, fs, ls):
        base = wid * RPW + j * RC
        pltpu.async_copy(feat_hbm.at[pl.ds(base, RC)], fv, fs)
        pltpu.async_copy(lab_hbm.at[pl.ds(base, RC)], lv, ls)

    def waitf(fv, lv, fs, ls):
        pltpu.make_async_copy(feat_hbm.at[pl.ds(0, RC)], fv, fs).wait()
        pltpu.make_async_copy(lab_hbm.at[pl.ds(0, RC)], lv, ls).wait()

    def process(j, fv, lv):
        pltpu.async_copy(means_hbm.at[lv], mean_v, sg).wait()

        def row(r, vec):
            def colstep(k2, acc):
                for u in range(4):
                    sl = pl.ds(k2 * 64 + u * 16, 16)
                    dd = fv[r, sl] - mean_v[r, sl]
                    acc = acc + dd * dd
                return acc

            acc = lax.fori_loop(0, D // 64, colstep,
                                jnp.zeros((16,), jnp.float32))
            return jnp.where(lanes == r, jnp.sum(acc), vec)

        vec = lax.fori_loop(0, RC, row, jnp.zeros((16,), jnp.float32))
        ssq_v[pl.ds(j * RC, RC)] = vec

    npairs = RPW // RC // 2
    startf(0, feat0, lab0, sf0, sl0)

    def pairstep(p, _):
        j0 = 2 * p
        waitf(feat0, lab0, sf0, sl0)
        startf(j0 + 1, feat1, lab1, sf1, sl1)
        process(j0, feat0, lab0)
        waitf(feat1, lab1, sf1, sl1)

        @pl.when(p < npairs - 1)
        def _():
            startf(j0 + 2, feat0, lab0, sf0, sl0)

        process(j0 + 1, feat1, lab1)
        return 0

    lax.fori_loop(0, npairs, pairstep, 0)
    pltpu.sync_copy(ssq_v, out_hbm.at[pl.ds(wid * RPW, RPW)])


def _topk_body(lab_ref, ssq_ref, idx_ref, w_ref):
    i = pl.program_id(0)
    lab = lab_ref[...]
    d = jnp.sqrt(ssq_ref[...] + 1e-12)
    cls = i * CBL + lax.broadcasted_iota(jnp.int32, (CBL, 1), 0)
    member = lab == cls
    cnt = jnp.sum(member.astype(jnp.int32), axis=1, keepdims=True)
    cntf = jnp.maximum(cnt, 1).astype(jnp.float32)
    dist = jnp.where(member, d, jnp.inf)
    col = lax.broadcasted_iota(jnp.int32, (CBL, N), 1)
    col128 = lax.broadcasted_iota(jnp.int32, (CBL, 128), 1)
    idx_mat = jnp.zeros((CBL, 128), jnp.int32)
    w_mat = jnp.zeros((CBL, 128), jnp.float32)
    for k in range(K):
        m = jnp.min(dist, axis=1, keepdims=True)
        am = jnp.min(jnp.where(dist == m, col, N), axis=1, keepdims=True)
        wk = jnp.where(cnt >= K, jnp.float32(1.0 / K),
                       jnp.where(k < cnt, 1.0 / cntf, jnp.float32(0.0)))
        if k == 0:
            idx_mat = jnp.broadcast_to(am, (CBL, 128)).astype(jnp.int32)
        else:
            idx_mat = jnp.where(col128 == k, am, idx_mat)
        w_mat = jnp.where(col128 == k, wk, w_mat)
        dist = jnp.where(col == am, jnp.inf, dist)
    idx_ref[...] = idx_mat
    w_ref[...] = w_mat


def _topk_call(lab1, ssq1):
    return pl.pallas_call(
        _topk_body,
        grid=(CPAD // CBL,),
        in_specs=[
            pl.BlockSpec((1, N), lambda i: (0, 0)),
            pl.BlockSpec((1, N), lambda i: (0, 0)),
        ],
        out_specs=[
            pl.BlockSpec((CBL, 128), lambda i: (i, 0)),
            pl.BlockSpec((CBL, 128), lambda i: (i, 0)),
        ],
        out_shape=[
            jax.ShapeDtypeStruct((CPAD, 128), jnp.int32),
            jax.ShapeDtypeStruct((CPAD, 128), jnp.float32),
        ],
    )(lab1, ssq1)


def _protos_body(feat_hbm, idx_hbm, w_hbm, out_hbm,
                 idx_v, w_v, rows0, rows1, proto_v, sem0, sem1):
    cidx = lax.axis_index("c")
    s = lax.axis_index("s")
    wid = s * NC + cidx
    base = wid * CPT
    pltpu.sync_copy(idx_hbm.at[pl.ds(base, CPT)], idx_v)
    pltpu.sync_copy(w_hbm.at[pl.ds(base, CPT)], w_v)

    def startg(t, rv, sem):
        pltpu.async_copy(feat_hbm.at[idx_v.at[t, pl.ds(0, 8)]], rv, sem)

    def waitg(rv, sem):
        pltpu.make_async_copy(feat_hbm.at[pl.ds(0, 8)], rv, sem).wait()

    def compute(t, rv):
        wvec = w_v[t, pl.ds(0, 16)]
        w0 = wvec[0]
        w1 = wvec[1]
        w2 = wvec[2]
        w3 = wvec[3]
        w4 = wvec[4]

        def col(kk, _):
            sl = pl.ds(kk * 16, 16)
            acc = rv[0, sl] * w0
            acc = acc + rv[1, sl] * w1
            acc = acc + rv[2, sl] * w2
            acc = acc + rv[3, sl] * w3
            acc = acc + rv[4, sl] * w4
            proto_v[t, sl] = acc
            return 0

        lax.fori_loop(0, D // 16, col, 0)

    npairs = CPT // 2
    startg(0, rows0, sem0)

    def pairstep(p, _):
        t0 = 2 * p
        waitg(rows0, sem0)
        startg(t0 + 1, rows1, sem1)
        compute(t0, rows0)
        waitg(rows1, sem1)

        @pl.when(p < npairs - 1)
        def _():
            startg(t0 + 2, rows0, sem0)

        compute(t0 + 1, rows1)
        return 0

    lax.fori_loop(0, npairs, pairstep, 0)
    pltpu.sync_copy(proto_v, out_hbm.at[pl.ds(base, CPT)])


@functools.lru_cache(maxsize=1)
def _build_sc_kernels():
    mesh = plsc.VectorSubcoreMesh(
        core_axis_name="c", subcore_axis_name="s",
        num_cores=NC, num_subcores=NS)
    params = pltpu.CompilerParams(needs_layout_passes=False)
    segment_sums = pl.kernel(
        _segment_sums_body,
        out_type=jax.ShapeDtypeStruct((NW, CPAD * TCOLS), jnp.float32),
        mesh=mesh,
        compiler_params=params,
        scratch_types=[
            pltpu.VMEM((RA, 128), jnp.float32),
            pltpu.VMEM((RA,), jnp.int32),
            pltpu.VMEM((RA, 128), jnp.float32),
            pltpu.VMEM((RA,), jnp.int32),
            pltpu.VMEM((CPAD * TCOLS,), jnp.float32),
            pltpu.SemaphoreType.DMA,
            pltpu.SemaphoreType.DMA,
            pltpu.SemaphoreType.DMA,
            pltpu.SemaphoreType.DMA,
        ],
    )
    dists = pl.kernel(
        _dists_body,
        out_type=jax.ShapeDtypeStruct((N,), jnp.float32),
        mesh=mesh,
        compiler_params=params,
        scratch_types=[
            pltpu.VMEM((RC, D), jnp.float32),
            pltpu.VMEM((RC, D), jnp.float32),
            pltpu.VMEM((RC,), jnp.int32),
            pltpu.VMEM((RC,), jnp.int32),
            pltpu.VMEM((RC, D), jnp.float32),
            pltpu.VMEM((RPW,), jnp.float32),
            pltpu.SemaphoreType.DMA,
            pltpu.SemaphoreType.DMA,
            pltpu.SemaphoreType.DMA,
            pltpu.SemaphoreType.DMA,
            pltpu.SemaphoreType.DMA,
        ],
    )
    protos_k = pl.kernel(
        _protos_body,
        out_type=jax.ShapeDtypeStruct((CPAD, D), jnp.float32),
        mesh=mesh,
        compiler_params=params,
        scratch_types=[
            pltpu.VMEM((CPT, 128), jnp.int32),
            pltpu.VMEM((CPT, 128), jnp.float32),
            pltpu.VMEM((8, D), jnp.float32),
            pltpu.VMEM((8, D), jnp.float32),
            pltpu.VMEM((CPT, D), jnp.float32),
            pltpu.SemaphoreType.DMA,
            pltpu.SemaphoreType.DMA,
        ],
    )
    return segment_sums, dists, protos_k


def kernel(features, labels):
    segment_sums, dists, protos_k = _build_sc_kernels()
    labels = labels.astype(jnp.int32)
    lab8 = labels.reshape(8, N // 8)
    sums3 = segment_sums(features, labels)
    sums = sums3.reshape(NW, CPAD, TCOLS).transpose(1, 0, 2).reshape(CPAD, D)
    means = _means_call(lab8, sums)
    ssq = dists(features, labels, means)
    idx128, w128 = _topk_call(labels.reshape(1, N), ssq.reshape(1, N))
    protos = protos_k(features, idx128, w128)
    return protos[:C]


# speedup vs baseline: 4.9883x; 1.0008x over previous
import functools

import jax
import jax.numpy as jnp
from jax import lax
from jax.experimental import pallas as pl
from jax.experimental.pallas import tpu as pltpu
from jax.experimental.pallas import tpu_sc as plsc

C = 1074
K = 5
D = 2048
N = 16384

NC, NS = 2, 16
NW = NC * NS
CPAD = 1280
CPT = CPAD // NW
SPC = CPAD // NS
DH = D // NC
RPS = N // NS
RA = 128
TCOLS = D // NW
RPW = N // NW
RC = 16
CB = 8
CBL = 128

def _segment_sums_body(feat_hbm, lab_hbm, out_hbm, feat_v, lab_v,
                       feat_v1, lab_v1, acc_v, sem_f0, sem_l0, sem_f1, sem_l1):
    cidx = lax.axis_index("c")
    s = lax.axis_index("s")
    wid = s * NC + cidx
    half = (wid % 2) * TCOLS

    def z(i, _):
        acc_v[pl.ds(i * 16, 16)] = jnp.zeros((16,), jnp.float32)
        return 0

    lax.fori_loop(0, CPAD * TCOLS // 16, z, 0)

    slab = (wid // 2) * 128

    def start(jc, fv, lv, fs, ls):
        rbase = jc * RA
        pltpu.async_copy(feat_hbm.at[pl.ds(rbase, RA), pl.ds(slab, 128)], fv, fs)
        pltpu.async_copy(lab_hbm.at[pl.ds(rbase, RA)], lv, ls)

    def wait(fv, lv, fs, ls):
        pltpu.make_async_copy(feat_hbm.at[pl.ds(0, RA), pl.ds(slab, 128)], fv, fs).wait()
        pltpu.make_async_copy(lab_hbm.at[pl.ds(0, RA)], lv, ls).wait()

    def process(fv, lv):
        @plsc.parallel_loop(0, RA // 16)
        def grp(q):
            lvec = lv[pl.ds(q * 16, 16)]
            bases = [lvec[rr] * TCOLS for rr in range(16)]
            for rr in range(16):
                for u in range(TCOLS // 16):
                    x = fv[q * 16 + rr, pl.ds(half + u * 16, 16)]
                    plsc.addupdate(acc_v.at[pl.ds(bases[rr] + u * 16, 16)], x)

    npairs = N // RA // 2
    start(0, feat_v, lab_v, sem_f0, sem_l0)

    def pairstep(p, _):
        jc0 = p * 2
        wait(feat_v, lab_v, sem_f0, sem_l0)
        start(jc0 + 1, feat_v1, lab_v1, sem_f1, sem_l1)
        process(feat_v, lab_v)
        wait(feat_v1, lab_v1, sem_f1, sem_l1)

        @pl.when(p < npairs - 1)
        def _():
            start(jc0 + 2, feat_v, lab_v, sem_f0, sem_l0)

        process(feat_v1, lab_v1)
        return 0

    lax.fori_loop(0, npairs, pairstep, 0)
    pltpu.sync_copy(acc_v, out_hbm.at[wid])


def _means_body(lab_ref, sums_ref, means_ref):
    i = pl.program_id(0)
    lab = lab_ref[...]
    for t in range(CB):
        cid = i * CB + t
        cnt = jnp.sum(jnp.where(lab == cid, 1, 0))
        denom = jnp.maximum(cnt, 1).astype(jnp.float32)
        means_ref[pl.ds(t, 1), :] = sums_ref[pl.ds(t, 1), :] / denom


def _means_call(lab8, sums):
    return pl.pallas_call(
        _means_body,
        grid=(CPAD // CB,),
        in_specs=[
            pl.BlockSpec((8, N // 8), lambda i: (0, 0)),
            pl.BlockSpec((CB, D), lambda i: (i, 0)),
        ],
        out_specs=pl.BlockSpec((CB, D), lambda i: (i, 0)),
        out_shape=jax.ShapeDtypeStruct((CPAD, D), jnp.float32),
    )(lab8, sums)


def _dists_body(feat_hbm, lab_hbm, means_hbm, out_hbm,
                feat0, feat1, lab0, lab1, mean_v, ssq_v,
                sf0, sf1, sl0, sl1, sg):
    cidx = lax.axis_index("c")
    s = lax.axis_index("s")
    wid = s * NC + cidx

    lanes = lax.iota(jnp.int32, 16)

    def startf(j, fv, lv, fs, ls):
        base = wid * RPW + j * RC
        pltpu.async_copy(feat_hbm.at[pl.ds(base, RC)], fv, fs)
        pltpu.async_copy(lab_hbm.at[pl.ds(base, RC)], lv, ls)

    def waitf(fv, lv, fs, ls):
        pltpu.make_async_copy(feat_hbm.at[pl.ds(0, RC)], fv, fs).wait()
        pltpu.make_async_copy(lab_hbm.at[pl.ds(0, RC)], lv, ls).wait()

    def process(j, fv, lv):
        pltpu.async_copy(means_hbm.at[lv], mean_v, sg).wait()

        def row(r, vec):
            def colstep(k2, acc):
                for u in range(4):
                    sl = pl.ds(k2 * 64 + u * 16, 16)
                    dd = fv[r, sl] - mean_v[r, sl]
                    acc = acc + dd * dd
                return acc

            acc = lax.fori_loop(0, D // 64, colstep,
                                jnp.zeros((16,), jnp.float32))
            return jnp.where(lanes == r, jnp.sum(acc), vec)

        vec = plsc.parallel_loop(
            0, RC, carry=jnp.zeros((16,), jnp.float32))(row)
        ssq_v[pl.ds(j * RC, RC)] = vec

    npairs = RPW // RC // 2
    startf(0, feat0, lab0, sf0, sl0)

    def pairstep(p, _):
        j0 = 2 * p
        waitf(feat0, lab0, sf0, sl0)
        startf(j0 + 1, feat1, lab1, sf1, sl1)
        process(j0, feat0, lab0)
        waitf(feat1, lab1, sf1, sl1)

        @pl.when(p < npairs - 1)
        def _():
            startf(j0 + 2, feat0, lab0, sf0, sl0)

        process(j0 + 1, feat1, lab1)
        return 0

    lax.fori_loop(0, npairs, pairstep, 0)
    pltpu.sync_copy(ssq_v, out_hbm.at[pl.ds(wid * RPW, RPW)])


def _topk_body(lab_ref, ssq_ref, idx_ref, w_ref):
    i = pl.program_id(0)
    lab = lab_ref[...]
    d = jnp.sqrt(ssq_ref[...] + 1e-12)
    cls = i * CBL + lax.broadcasted_iota(jnp.int32, (CBL, 1), 0)
    member = lab == cls
    cnt = jnp.sum(member.astype(jnp.int32), axis=1, keepdims=True)
    cntf = jnp.maximum(cnt, 1).astype(jnp.float32)
    dist = jnp.where(member, d, jnp.inf)
    col = lax.broadcasted_iota(jnp.int32, (CBL, N), 1)
    col128 = lax.broadcasted_iota(jnp.int32, (CBL, 128), 1)
    idx_mat = jnp.zeros((CBL, 128), jnp.int32)
    w_mat = jnp.zeros((CBL, 128), jnp.float32)
    for k in range(K):
        m = jnp.min(dist, axis=1, keepdims=True)
        am = jnp.min(jnp.where(dist == m, col, N), axis=1, keepdims=True)
        wk = jnp.where(cnt >= K, jnp.float32(1.0 / K),
                       jnp.where(k < cnt, 1.0 / cntf, jnp.float32(0.0)))
        if k == 0:
            idx_mat = jnp.broadcast_to(am, (CBL, 128)).astype(jnp.int32)
        else:
            idx_mat = jnp.where(col128 == k, am, idx_mat)
        w_mat = jnp.where(col128 == k, wk, w_mat)
        dist = jnp.where(col == am, jnp.inf, dist)
    idx_ref[...] = idx_mat
    w_ref[...] = w_mat


def _topk_call(lab1, ssq1):
    return pl.pallas_call(
        _topk_body,
        grid=(CPAD // CBL,),
        in_specs=[
            pl.BlockSpec((1, N), lambda i: (0, 0)),
            pl.BlockSpec((1, N), lambda i: (0, 0)),
        ],
        out_specs=[
            pl.BlockSpec((CBL, 128), lambda i: (i, 0)),
            pl.BlockSpec((CBL, 128), lambda i: (i, 0)),
        ],
        out_shape=[
            jax.ShapeDtypeStruct((CPAD, 128), jnp.int32),
            jax.ShapeDtypeStruct((CPAD, 128), jnp.float32),
        ],
    )(lab1, ssq1)


def _protos_body(feat_hbm, idx_hbm, w_hbm, out_hbm,
                 idx_v, w_v, rows0, rows1, proto_v, sem0, sem1):
    cidx = lax.axis_index("c")
    s = lax.axis_index("s")
    wid = s * NC + cidx
    base = wid * CPT
    pltpu.sync_copy(idx_hbm.at[pl.ds(base, CPT)], idx_v)
    pltpu.sync_copy(w_hbm.at[pl.ds(base, CPT)], w_v)

    def startg(t, rv, sem):
        pltpu.async_copy(feat_hbm.at[idx_v.at[t, pl.ds(0, 8)]], rv, sem)

    def waitg(rv, sem):
        pltpu.make_async_copy(feat_hbm.at[pl.ds(0, 8)], rv, sem).wait()

    def compute(t, rv):
        wvec = w_v[t, pl.ds(0, 16)]
        w0 = wvec[0]
        w1 = wvec[1]
        w2 = wvec[2]
        w3 = wvec[3]
        w4 = wvec[4]

        @plsc.parallel_loop(0, D // 16)
        def col(kk):
            sl = pl.ds(kk * 16, 16)
            acc = rv[0, sl] * w0
            acc = acc + rv[1, sl] * w1
            acc = acc + rv[2, sl] * w2
            acc = acc + rv[3, sl] * w3
            acc = acc + rv[4, sl] * w4
            proto_v[t, sl] = acc

    npairs = CPT // 2
    startg(0, rows0, sem0)

    def pairstep(p, _):
        t0 = 2 * p
        waitg(rows0, sem0)
        startg(t0 + 1, rows1, sem1)
        compute(t0, rows0)
        waitg(rows1, sem1)

        @pl.when(p < npairs - 1)
        def _():
            startg(t0 + 2, rows0, sem0)

        compute(t0 + 1, rows1)
        return 0

    lax.fori_loop(0, npairs, pairstep, 0)
    pltpu.sync_copy(proto_v, out_hbm.at[pl.ds(base, CPT)])


@functools.lru_cache(maxsize=1)
def _build_sc_kernels():
    mesh = plsc.VectorSubcoreMesh(
        core_axis_name="c", subcore_axis_name="s",
        num_cores=NC, num_subcores=NS)
    params = pltpu.CompilerParams(needs_layout_passes=False)
    segment_sums = pl.kernel(
        _segment_sums_body,
        out_type=jax.ShapeDtypeStruct((NW, CPAD * TCOLS), jnp.float32),
        mesh=mesh,
        compiler_params=params,
        scratch_types=[
            pltpu.VMEM((RA, 128), jnp.float32),
            pltpu.VMEM((RA,), jnp.int32),
            pltpu.VMEM((RA, 128), jnp.float32),
            pltpu.VMEM((RA,), jnp.int32),
            pltpu.VMEM((CPAD * TCOLS,), jnp.float32),
            pltpu.SemaphoreType.DMA,
            pltpu.SemaphoreType.DMA,
            pltpu.SemaphoreType.DMA,
            pltpu.SemaphoreType.DMA,
        ],
    )
    dists = pl.kernel(
        _dists_body,
        out_type=jax.ShapeDtypeStruct((N,), jnp.float32),
        mesh=mesh,
        compiler_params=params,
        scratch_types=[
            pltpu.VMEM((RC, D), jnp.float32),
            pltpu.VMEM((RC, D), jnp.float32),
            pltpu.VMEM((RC,), jnp.int32),
            pltpu.VMEM((RC,), jnp.int32),
            pltpu.VMEM((RC, D), jnp.float32),
            pltpu.VMEM((RPW,), jnp.float32),
            pltpu.SemaphoreType.DMA,
            pltpu.SemaphoreType.DMA,
            pltpu.SemaphoreType.DMA,
            pltpu.SemaphoreType.DMA,
            pltpu.SemaphoreType.DMA,
        ],
    )
    protos_k = pl.kernel(
        _protos_body,
        out_type=jax.ShapeDtypeStruct((CPAD, D), jnp.float32),
        mesh=mesh,
        compiler_params=params,
        scratch_types=[
            pltpu.VMEM((CPT, 128), jnp.int32),
            pltpu.VMEM((CPT, 128), jnp.float32),
            pltpu.VMEM((8, D), jnp.float32),
            pltpu.VMEM((8, D), jnp.float32),
            pltpu.VMEM((CPT, D), jnp.float32),
            pltpu.SemaphoreType.DMA,
            pltpu.SemaphoreType.DMA,
        ],
    )
    return segment_sums, dists, protos_k


def kernel(features, labels):
    segment_sums, dists, protos_k = _build_sc_kernels()
    labels = labels.astype(jnp.int32)
    lab8 = labels.reshape(8, N // 8)
    sums3 = segment_sums(features, labels)
    sums = sums3.reshape(NW, CPAD, TCOLS).transpose(1, 0, 2).reshape(CPAD, D)
    means = _means_call(lab8, sums)
    ssq = dists(features, labels, means)
    idx128, w128 = _topk_call(labels.reshape(1, N), ssq.reshape(1, N))
    protos = protos_k(features, idx128, w128)
    return protos[:C]
